# baseline (speedup 1.0000x reference)
import functools

import jax
import jax.numpy as jnp
from jax.experimental import pallas as pl
from jax.experimental.pallas import tpu as pltpu

_M_TILE = 512
_N_TILE = 512
_K_TILE = 512
_ALPHA = 16.0
_RANK = 16
_SCALING = _ALPHA / _RANK


def _fused_kernel(x_ref, w_ref, a_ref, b_ref, o_ref, mid_ref, *, nk, scaling):
    j = pl.program_id(1)
    k = pl.program_id(2)

    @pl.when(k == 0)
    def _zero_out():
        o_ref[...] = jnp.zeros_like(o_ref)

    @pl.when(jnp.logical_and(j == 0, k == 0))
    def _zero_mid():
        mid_ref[...] = jnp.zeros_like(mid_ref)

    xb = x_ref[...]
    o_ref[...] += jax.lax.dot_general(
        xb, w_ref[...], (((1,), (1,)), ((), ())),
        preferred_element_type=jnp.float32)

    @pl.when(j == 0)
    def _acc_mid():
        mid_ref[...] += jax.lax.dot_general(
            xb, a_ref[...], (((1,), (1,)), ((), ())),
            preferred_element_type=jnp.float32)

    @pl.when(k == nk - 1)
    def _epilogue():
        o_ref[...] += scaling * jax.lax.dot_general(
            mid_ref[...], b_ref[...], (((1,), (1,)), ((), ())),
            preferred_element_type=jnp.float32)


@jax.jit
def kernel(x, weight, lora_A, lora_B):
    m, kdim = x.shape
    n = weight.shape[0]
    a0 = lora_A[0, :_RANK, :]
    b0 = lora_B[0, :, :_RANK]
    nk = kdim // _K_TILE
    grid = (m // _M_TILE, n // _N_TILE, nk)
    return pl.pallas_call(
        functools.partial(_fused_kernel, nk=nk, scaling=_SCALING),
        grid=grid,
        in_specs=[
            pl.BlockSpec((_M_TILE, _K_TILE), lambda i, j, k: (i, k)),
            pl.BlockSpec((_N_TILE, _K_TILE), lambda i, j, k: (j, k)),
            pl.BlockSpec((_RANK, _K_TILE), lambda i, j, k: (0, k)),
            pl.BlockSpec((_N_TILE, _RANK), lambda i, j, k: (j, 0)),
        ],
        out_specs=pl.BlockSpec((_M_TILE, _N_TILE), lambda i, j, k: (i, j)),
        out_shape=jax.ShapeDtypeStruct((m, n), jnp.float32),
        scratch_shapes=[pltpu.VMEM((_M_TILE, _RANK), jnp.float32)],
        compiler_params=pltpu.CompilerParams(
            dimension_semantics=("parallel", "arbitrary", "arbitrary"),
        ),
    )(x, weight, a0, b0)


# speedup vs baseline: 3.5809x; 3.5809x over previous
import functools

import jax
import jax.numpy as jnp
from jax.experimental import pallas as pl
from jax.experimental.pallas import tpu as pltpu

_ALPHA = 16.0
_RANK = 16
_SCALING = _ALPHA / _RANK

_FN = 512
_FK = 512
_BM = 2048
_BN = 2048
_BK = 512


def _fold_kernel(w_ref, b_ref, a_ref, o_ref, *, scaling):
    delta = jax.lax.dot_general(
        b_ref[...].astype(jnp.bfloat16), a_ref[...].astype(jnp.bfloat16),
        (((1,), (0,)), ((), ())), preferred_element_type=jnp.float32)
    o_ref[...] = (w_ref[...] + scaling * delta).astype(jnp.bfloat16)


def _matmul_kernel(x_ref, w_ref, o_ref, *, nk):
    k = pl.program_id(2)

    @pl.when(k == 0)
    def _zero():
        o_ref[...] = jnp.zeros_like(o_ref)

    o_ref[...] += jax.lax.dot_general(
        x_ref[...].astype(jnp.bfloat16), w_ref[...],
        (((1,), (1,)), ((), ())), preferred_element_type=jnp.float32)


@jax.jit
def kernel(x, weight, lora_A, lora_B):
    m, kdim = x.shape
    n = weight.shape[0]
    a0 = lora_A[0, :_RANK, :]
    b0 = lora_B[0, :, :_RANK]

    w_eff = pl.pallas_call(
        functools.partial(_fold_kernel, scaling=_SCALING),
        grid=(n // _FN, kdim // _FK),
        in_specs=[
            pl.BlockSpec((_FN, _FK), lambda j, k: (j, k)),
            pl.BlockSpec((_FN, _RANK), lambda j, k: (j, 0)),
            pl.BlockSpec((_RANK, _FK), lambda j, k: (0, k)),
        ],
        out_specs=pl.BlockSpec((_FN, _FK), lambda j, k: (j, k)),
        out_shape=jax.ShapeDtypeStruct((n, kdim), jnp.bfloat16),
        compiler_params=pltpu.CompilerParams(
            dimension_semantics=("parallel", "parallel"),
        ),
    )(weight, b0, a0)

    nk = kdim // _BK
    return pl.pallas_call(
        functools.partial(_matmul_kernel, nk=nk),
        grid=(m // _BM, n // _BN, nk),
        in_specs=[
            pl.BlockSpec((_BM, _BK), lambda i, j, k: (i, k)),
            pl.BlockSpec((_BN, _BK), lambda i, j, k: (j, k)),
        ],
        out_specs=pl.BlockSpec((_BM, _BN), lambda i, j, k: (i, j)),
        out_shape=jax.ShapeDtypeStruct((m, n), jnp.float32),
        compiler_params=pltpu.CompilerParams(
            dimension_semantics=("parallel", "parallel", "arbitrary"),
        ),
    )(x, w_eff)
